# baseline (speedup 1.0000x reference)
import functools

import jax
import jax.numpy as jnp
from jax import lax
from jax.experimental import pallas as pl
from jax.experimental.pallas import tpu as pltpu
from jax.experimental.pallas import tpu_sc as plsc


def _sc_info():
    try:
        info = plsc.get_sparse_core_info()
        return info.num_cores, info.num_subcores
    except Exception:
        return 2, 16


def _sc_gather_user(user_emb, u_idx):
    B = u_idx.shape[0]
    nc, ns = _sc_info()
    bpw = B // (nc * ns)
    mesh = plsc.VectorSubcoreMesh(core_axis_name="c", subcore_axis_name="s")

    @functools.partial(
        pl.kernel,
        mesh=mesh,
        out_type=jax.ShapeDtypeStruct((B, 128), jnp.float32),
        scratch_types=[
            pltpu.VMEM((bpw,), jnp.int32),
            pltpu.VMEM((bpw, 128), jnp.float32),
            pltpu.SemaphoreType.DMA,
        ],
    )
    def g(ue_hbm, ui_hbm, ou_hbm, ui_v, r_v, sem):
        wid = lax.axis_index("s") * nc + lax.axis_index("c")
        base = wid * bpw
        pltpu.sync_copy(ui_hbm.at[pl.ds(base, bpw)], ui_v)
        pltpu.async_copy(ue_hbm.at[ui_v], r_v, sem).wait()
        pltpu.sync_copy(r_v, ou_hbm.at[pl.ds(base, bpw)])

    return g(user_emb, u_idx)


def _sc_gather_small(small_tab, g_idx, a_idx, o_idx):
    B = g_idx.shape[0]
    nc, ns = _sc_info()
    bpw = B // (nc * ns)
    mesh = plsc.VectorSubcoreMesh(core_axis_name="c", subcore_axis_name="s")

    @functools.partial(
        pl.kernel,
        mesh=mesh,
        out_type=jax.ShapeDtypeStruct((B, 128), jnp.float32),
        scratch_types=[
            pltpu.VMEM((bpw,), jnp.int32),
            pltpu.VMEM((bpw,), jnp.int32),
            pltpu.VMEM((bpw,), jnp.int32),
            pltpu.VMEM((bpw, 128), jnp.float32),
            pltpu.SemaphoreType.DMA,
        ],
    )
    def g(ts_hbm, gi_hbm, ai_hbm, oi_hbm, os_hbm, gi_v, ai_v, oi_v, r_v, sem):
        wid = lax.axis_index("s") * nc + lax.axis_index("c")
        base = wid * bpw
        pltpu.sync_copy(gi_hbm.at[pl.ds(base, bpw)], gi_v)
        pltpu.sync_copy(ai_hbm.at[pl.ds(base, bpw)], ai_v)
        pltpu.sync_copy(oi_hbm.at[pl.ds(base, bpw)], oi_v)

        def body(i, _):
            s = pl.ds(i * 16, 16)
            gi_v[s] = (gi_v[s] * 7 + ai_v[s]) * 21 + oi_v[s]
            return 0

        lax.fori_loop(0, bpw // 16, body, 0)
        pltpu.async_copy(ts_hbm.at[gi_v], r_v, sem).wait()
        pltpu.sync_copy(r_v, os_hbm.at[pl.ds(base, bpw)])

    return g(small_tab, g_idx, a_idx, o_idx)


def _mlp_body(xu_ref, xs_ref,
              W1_ref, g1_ref, be1_ref,
              W2_ref, b2_ref, g2_ref, be2_ref,
              W3_ref, b3_ref, out_ref):
    x = jnp.concatenate([xu_ref[...], xs_ref[...]], axis=-1)

    h = jnp.dot(x, W1_ref[...], preferred_element_type=jnp.float32)
    mu = jnp.mean(h, axis=-1, keepdims=True)
    xc = h - mu
    var = jnp.mean(xc * xc, axis=-1, keepdims=True)
    h = xc * (jax.lax.rsqrt(var + 1e-5) * g1_ref[...]) + be1_ref[...]
    h = jnp.maximum(h, 0.0)

    h = jnp.dot(h, W2_ref[...], preferred_element_type=jnp.float32)
    h = h + b2_ref[...]
    mu = jnp.mean(h, axis=-1, keepdims=True)
    xc = h - mu
    var = jnp.mean(xc * xc, axis=-1, keepdims=True)
    h = xc * (jax.lax.rsqrt(var + 1e-5) * g2_ref[...]) + be2_ref[...]
    h = jnp.maximum(h, 0.0)

    z = jnp.dot(h, W3_ref[...], preferred_element_type=jnp.float32)
    z = z + b3_ref[...]
    n2 = jnp.sum(z * z, axis=-1, keepdims=True)
    out_ref[...] = z * jax.lax.rsqrt(jnp.maximum(n2, 1e-24))


def _mlp_call(BB, B, interpret=False):
    nb = B // BB

    def full(shape):
        return pl.BlockSpec(shape, lambda i: (0,) * len(shape))

    return pl.pallas_call(
        _mlp_body,
        grid=(nb,),
        in_specs=[
            pl.BlockSpec((BB, 128), lambda i: (i, 0)),
            pl.BlockSpec((BB, 128), lambda i: (i, 0)),
            full((256, 512)),
            full((1, 512)), full((1, 512)),
            full((512, 256)),
            full((1, 256)), full((1, 256)), full((1, 256)),
            full((256, 128)),
            full((1, 128)),
        ],
        out_specs=pl.BlockSpec((BB, 128), lambda i: (i, 0)),
        out_shape=jax.ShapeDtypeStruct((B, 128), jnp.float32),
        interpret=interpret,
    )


def _fuse_body(ge_ref, ae_ref, oe_ref, out_ref):
    j = jax.lax.broadcasted_iota(jnp.int32, (294, 1), 0)
    g = j // 147
    a = (j // 21) % 7
    o = j % 21
    ge = ge_ref[...]
    ae = ae_ref[...]
    oe = oe_ref[...]
    eg = jnp.where(g == 0, ge[0:1, :], ge[1:2, :])
    ea = jnp.where(a == 1, ae[1:2, :], ae[0:1, :])
    for r in range(2, 7):
        ea = jnp.where(a == r, ae[r:r + 1, :], ea)
    eo = jnp.where(o == 1, oe[1:2, :], oe[0:1, :])
    for r in range(2, 21):
        eo = jnp.where(o == r, oe[r:r + 1, :], eo)
    pad = jnp.zeros((294, 63), jnp.float32)
    one = jnp.ones((294, 1), jnp.float32)
    out_ref[...] = jnp.concatenate([eg, ea, eo, pad, one], axis=-1)


def _fuse_small_tables(gender_emb, age_emb, occ_emb):
    return pl.pallas_call(
        _fuse_body,
        out_shape=jax.ShapeDtypeStruct((294, 128), jnp.float32),
    )(gender_emb, age_emb, occ_emb)


def kernel(user_row, gender_idx, age_idx, occ_idx, user_emb, gender_emb,
           age_emb, occ_emb, W1, b1, g1, be1, W2, b2, g2, be2, W3, b3):
    B = user_row.shape[0]
    small_tab = _fuse_small_tables(gender_emb, age_emb, occ_emb)
    W1p = jnp.concatenate(
        [W1, jnp.zeros((63, 512), jnp.float32), b1.reshape(1, 512)], axis=0)
    xu = _sc_gather_user(user_emb, user_row.astype(jnp.int32))
    xs = _sc_gather_small(
        small_tab, gender_idx.astype(jnp.int32),
        age_idx.astype(jnp.int32), occ_idx.astype(jnp.int32))
    BB = 2048
    return _mlp_call(BB, B)(
        xu, xs,
        W1p, g1.reshape(1, 512), be1.reshape(1, 512),
        W2, b2.reshape(1, 256), g2.reshape(1, 256), be2.reshape(1, 256),
        W3, b3.reshape(1, 128),
    )


# speedup vs baseline: 1.0668x; 1.0668x over previous
import functools

import jax
import jax.numpy as jnp
from jax import lax
from jax.experimental import pallas as pl
from jax.experimental.pallas import tpu as pltpu
from jax.experimental.pallas import tpu_sc as plsc


def _sc_gather(user_emb, small_tab, u_idx, g_idx, a_idx, o_idx):
    B = u_idx.shape[0]
    try:
        info = plsc.get_sparse_core_info()
        nc, ns = info.num_cores, info.num_subcores
    except Exception:
        nc, ns = 2, 16
    nw = nc * ns
    bpw = B // nw
    mesh = plsc.VectorSubcoreMesh(core_axis_name="c", subcore_axis_name="s")

    @functools.partial(
        pl.kernel,
        mesh=mesh,
        out_type=(jax.ShapeDtypeStruct((B, 128), jnp.float32),
                  jax.ShapeDtypeStruct((B, 128), jnp.float32)),
        scratch_types=[
            pltpu.VMEM((bpw,), jnp.int32),
            pltpu.VMEM((bpw,), jnp.int32),
            pltpu.VMEM((bpw,), jnp.int32),
            pltpu.VMEM((bpw,), jnp.int32),
            pltpu.VMEM((bpw,), jnp.int32),
            pltpu.VMEM((bpw, 128), jnp.float32),
            pltpu.SemaphoreType.DMA,
        ],
    )
    def g(ue_hbm, ts_hbm, ui_hbm, gi_hbm, ai_hbm, oi_hbm,
          ou_hbm, os_hbm, ui_v, gi_v, ai_v, oi_v, ji_v, r_v, sem):
        wid = lax.axis_index("s") * nc + lax.axis_index("c")
        base = wid * bpw
        pltpu.sync_copy(ui_hbm.at[pl.ds(base, bpw)], ui_v)
        cu = pltpu.async_copy(ue_hbm.at[ui_v], r_v, sem)
        pltpu.sync_copy(gi_hbm.at[pl.ds(base, bpw)], gi_v)
        pltpu.sync_copy(ai_hbm.at[pl.ds(base, bpw)], ai_v)
        pltpu.sync_copy(oi_hbm.at[pl.ds(base, bpw)], oi_v)

        def body(i, _):
            s = pl.ds(i * 16, 16)
            ji_v[s] = (gi_v[s] * 7 + ai_v[s]) * 21 + oi_v[s]
            return 0

        lax.fori_loop(0, bpw // 16, body, 0)
        cu.wait()
        pltpu.sync_copy(r_v, ou_hbm.at[pl.ds(base, bpw)])
        pltpu.async_copy(ts_hbm.at[ji_v], r_v, sem).wait()
        pltpu.sync_copy(r_v, os_hbm.at[pl.ds(base, bpw)])

    return g(user_emb, small_tab, u_idx, g_idx, a_idx, o_idx)


def _mlp_body(xu_ref, xs_ref,
              W1_ref, g1_ref, be1_ref,
              W2_ref, b2_ref, g2_ref, be2_ref,
              W3_ref, b3_ref, out_ref):
    x = jnp.concatenate([xu_ref[...], xs_ref[...]], axis=-1)

    h = jnp.dot(x, W1_ref[...], preferred_element_type=jnp.float32)
    mu = jnp.mean(h, axis=-1, keepdims=True)
    xc = h - mu
    var = jnp.mean(xc * xc, axis=-1, keepdims=True)
    h = xc * (jax.lax.rsqrt(var + 1e-5) * g1_ref[...]) + be1_ref[...]
    h = jnp.maximum(h, 0.0)

    h = jnp.dot(h, W2_ref[...], preferred_element_type=jnp.float32)
    h = h + b2_ref[...]
    mu = jnp.mean(h, axis=-1, keepdims=True)
    xc = h - mu
    var = jnp.mean(xc * xc, axis=-1, keepdims=True)
    h = xc * (jax.lax.rsqrt(var + 1e-5) * g2_ref[...]) + be2_ref[...]
    h = jnp.maximum(h, 0.0)

    z = jnp.dot(h, W3_ref[...], preferred_element_type=jnp.float32)
    z = z + b3_ref[...]
    n2 = jnp.sum(z * z, axis=-1, keepdims=True)
    out_ref[...] = z * jax.lax.rsqrt(jnp.maximum(n2, 1e-24))


def _mlp_call(BB, B):
    nb = B // BB

    def full(shape):
        return pl.BlockSpec(shape, lambda i: (0,) * len(shape))

    return pl.pallas_call(
        _mlp_body,
        grid=(nb,),
        in_specs=[
            pl.BlockSpec((BB, 128), lambda i: (i, 0)),
            pl.BlockSpec((BB, 128), lambda i: (i, 0)),
            full((256, 512)),
            full((1, 512)), full((1, 512)),
            full((512, 256)),
            full((1, 256)), full((1, 256)), full((1, 256)),
            full((256, 128)),
            full((1, 128)),
        ],
        out_specs=pl.BlockSpec((BB, 128), lambda i: (i, 0)),
        out_shape=jax.ShapeDtypeStruct((B, 128), jnp.float32),
    )


def _fuse_body(ge_ref, ae_ref, oe_ref, out_ref):
    j = jax.lax.broadcasted_iota(jnp.int32, (294, 1), 0)
    g = j // 147
    a = (j // 21) % 7
    o = j % 21
    ge = ge_ref[...]
    ae = ae_ref[...]
    oe = oe_ref[...]
    eg = jnp.where(g == 0, ge[0:1, :], ge[1:2, :])
    ea = jnp.where(a == 1, ae[1:2, :], ae[0:1, :])
    for r in range(2, 7):
        ea = jnp.where(a == r, ae[r:r + 1, :], ea)
    eo = jnp.where(o == 1, oe[1:2, :], oe[0:1, :])
    for r in range(2, 21):
        eo = jnp.where(o == r, oe[r:r + 1, :], eo)
    pad = jnp.zeros((294, 63), jnp.float32)
    one = jnp.ones((294, 1), jnp.float32)
    out_ref[...] = jnp.concatenate([eg, ea, eo, pad, one], axis=-1)


def _fuse_small_tables(gender_emb, age_emb, occ_emb):
    return pl.pallas_call(
        _fuse_body,
        out_shape=jax.ShapeDtypeStruct((294, 128), jnp.float32),
    )(gender_emb, age_emb, occ_emb)


def kernel(user_row, gender_idx, age_idx, occ_idx, user_emb, gender_emb,
           age_emb, occ_emb, W1, b1, g1, be1, W2, b2, g2, be2, W3, b3):
    B = user_row.shape[0]
    small_tab = _fuse_small_tables(gender_emb, age_emb, occ_emb)
    W1p = jnp.concatenate(
        [W1, jnp.zeros((63, 512), jnp.float32), b1.reshape(1, 512)], axis=0)
    xu, xs = _sc_gather(
        user_emb, small_tab,
        user_row.astype(jnp.int32), gender_idx.astype(jnp.int32),
        age_idx.astype(jnp.int32), occ_idx.astype(jnp.int32))
    BB = 2048
    return _mlp_call(BB, B)(
        xu, xs,
        W1p, g1.reshape(1, 512), be1.reshape(1, 512),
        W2, b2.reshape(1, 256), g2.reshape(1, 256), be2.reshape(1, 256),
        W3, b3.reshape(1, 128),
    )
